# baseline (speedup 1.0000x reference)
import functools

import jax
import jax.numpy as jnp
from jax import lax
from jax.experimental import pallas as pl
from jax.experimental.pallas import tpu as pltpu
from jax.experimental.pallas import tpu_sc as plsc

VOCAB = 100000
EMBED = 128
HIDDEN = 1024
OUT = 45
WINDOW = 7
BATCH = 4096
FLAT = BATCH * WINDOW
NUM_WORKERS = 32
BPW = FLAT // NUM_WORKERS
CHUNK = 128
NCHUNK = BPW // CHUNK

BM = 512


_sc_mesh = plsc.VectorSubcoreMesh(core_axis_name="c", subcore_axis_name="s")


@functools.partial(
    pl.kernel,
    mesh=_sc_mesh,
    out_type=jax.ShapeDtypeStruct((FLAT, EMBED), jnp.float32),
    scratch_types=[
        pltpu.VMEM((NCHUNK, CHUNK), jnp.int32),
        pltpu.VMEM((BPW, EMBED), jnp.float32),
        pltpu.SemaphoreType.DMA,
        pltpu.SemaphoreType.DMA,
    ],
)
def _sc_gather(idx_hbm, table_hbm, out_hbm, idx_v, rows_v, gsem, osem):
    wid = lax.axis_index("s") * 2 + lax.axis_index("c")
    base = wid * BPW
    pltpu.sync_copy(idx_hbm.at[wid], idx_v)
    gathers = []
    for j in range(NCHUNK):
        gathers.append(
            pltpu.async_copy(
                table_hbm.at[idx_v.at[j]],
                rows_v.at[pl.ds(j * CHUNK, CHUNK)],
                gsem,
            )
        )
    for cp in gathers:
        cp.wait()
    pltpu.async_copy(rows_v, out_hbm.at[pl.ds(base, BPW)], osem).wait()


def _mlp_body(x_ref, w1_ref, b1_ref, w2_ref, b2_ref, o_ref, w1bf):
    @pl.when(pl.program_id(0) == 0)
    def _():
        w1bf[...] = w1_ref[...].astype(jnp.bfloat16)

    h = jnp.dot(
        x_ref[0].astype(jnp.bfloat16), w1bf[0],
        preferred_element_type=jnp.float32,
    )
    for p in range(1, WINDOW):
        h = h + jnp.dot(
            x_ref[p].astype(jnp.bfloat16), w1bf[p],
            preferred_element_type=jnp.float32,
        )
    h = jnp.maximum(h + b1_ref[...], 0.0).astype(jnp.bfloat16)
    w2 = w2_ref[...].astype(jnp.bfloat16)
    logits = jnp.dot(h, w2, preferred_element_type=jnp.float32)
    logits = logits + b2_ref[...]
    m = jnp.max(logits, axis=1, keepdims=True)
    lse = jnp.log(jnp.sum(jnp.exp(logits - m), axis=1, keepdims=True)) + m
    o_ref[...] = logits - lse


_mlp = pl.pallas_call(
    _mlp_body,
    grid=(BATCH // BM,),
    in_specs=[
        pl.BlockSpec((WINDOW, BM, EMBED), lambda i: (0, i, 0)),
        pl.BlockSpec((WINDOW, EMBED, HIDDEN), lambda i: (0, 0, 0)),
        pl.BlockSpec((1, HIDDEN), lambda i: (0, 0)),
        pl.BlockSpec((HIDDEN, OUT), lambda i: (0, 0)),
        pl.BlockSpec((1, OUT), lambda i: (0, 0)),
    ],
    out_specs=pl.BlockSpec((BM, OUT), lambda i: (i, 0)),
    out_shape=jax.ShapeDtypeStruct((BATCH, OUT), jnp.float32),
    scratch_shapes=[pltpu.VMEM((WINDOW, EMBED, HIDDEN), jnp.bfloat16)],
)


def kernel(inputs, embedding, W1, b1, W2, b2):
    idx = inputs.T.reshape(NUM_WORKERS, NCHUNK, CHUNK).astype(jnp.int32)
    gathered = _sc_gather(idx, embedding)
    x = gathered.reshape(WINDOW, BATCH, EMBED)
    w1 = W1.reshape(WINDOW, EMBED, HIDDEN)
    return _mlp(x, w1, b1.reshape(1, HIDDEN), W2, b2.reshape(1, OUT))


# speedup vs baseline: 1.0169x; 1.0169x over previous
import functools

import jax
import jax.numpy as jnp
from jax import lax
from jax.experimental import pallas as pl
from jax.experimental.pallas import tpu as pltpu
from jax.experimental.pallas import tpu_sc as plsc

VOCAB = 100000
EMBED = 128
HIDDEN = 1024
OUT = 45
WINDOW = 7
BATCH = 4096
FLAT = BATCH * WINDOW
NUM_WORKERS = 32
BPW = FLAT // NUM_WORKERS
CHUNK = 128
NCHUNK = BPW // CHUNK

BM = 2048


_sc_mesh = plsc.VectorSubcoreMesh(core_axis_name="c", subcore_axis_name="s")


@functools.partial(
    pl.kernel,
    mesh=_sc_mesh,
    out_type=jax.ShapeDtypeStruct((FLAT, EMBED), jnp.float32),
    scratch_types=[
        pltpu.VMEM((NCHUNK, CHUNK), jnp.int32),
        pltpu.VMEM((BPW, EMBED), jnp.float32),
        pltpu.SemaphoreType.DMA,
        pltpu.SemaphoreType.DMA,
    ],
)
def _sc_gather(idx_hbm, table_hbm, out_hbm, idx_v, rows_v, gsem, osem):
    wid = lax.axis_index("s") * 2 + lax.axis_index("c")
    base = wid * BPW
    pltpu.sync_copy(idx_hbm.at[wid], idx_v)
    gathers = []
    for j in range(NCHUNK):
        gathers.append(
            pltpu.async_copy(
                table_hbm.at[idx_v.at[j]],
                rows_v.at[pl.ds(j * CHUNK, CHUNK)],
                gsem,
            )
        )
    for cp in gathers:
        cp.wait()
    pltpu.async_copy(rows_v, out_hbm.at[pl.ds(base, BPW)], osem).wait()


def _mlp_body(x_ref, w1_ref, b1_ref, w2_ref, b2_ref, o_ref, w1bf):
    @pl.when(pl.program_id(0) == 0)
    def _():
        w1bf[...] = w1_ref[...].astype(jnp.bfloat16)

    h = jnp.dot(
        x_ref[0].astype(jnp.bfloat16), w1bf[0],
        preferred_element_type=jnp.float32,
    )
    for p in range(1, WINDOW):
        h = h + jnp.dot(
            x_ref[p].astype(jnp.bfloat16), w1bf[p],
            preferred_element_type=jnp.float32,
        )
    h = jnp.maximum(h + b1_ref[...], 0.0).astype(jnp.bfloat16)
    w2 = w2_ref[...].astype(jnp.bfloat16)
    logits = jnp.dot(h, w2, preferred_element_type=jnp.float32)
    logits = logits + b2_ref[...]
    m = jnp.max(logits, axis=1, keepdims=True)
    lse = jnp.log(jnp.sum(jnp.exp(logits - m), axis=1, keepdims=True)) + m
    o_ref[...] = logits - lse


_mlp = pl.pallas_call(
    _mlp_body,
    grid=(BATCH // BM,),
    in_specs=[
        pl.BlockSpec((WINDOW, BM, EMBED), lambda i: (0, i, 0)),
        pl.BlockSpec((WINDOW, EMBED, HIDDEN), lambda i: (0, 0, 0)),
        pl.BlockSpec((1, HIDDEN), lambda i: (0, 0)),
        pl.BlockSpec((HIDDEN, OUT), lambda i: (0, 0)),
        pl.BlockSpec((1, OUT), lambda i: (0, 0)),
    ],
    out_specs=pl.BlockSpec((BM, OUT), lambda i: (i, 0)),
    out_shape=jax.ShapeDtypeStruct((BATCH, OUT), jnp.float32),
    scratch_shapes=[pltpu.VMEM((WINDOW, EMBED, HIDDEN), jnp.bfloat16)],
)


def kernel(inputs, embedding, W1, b1, W2, b2):
    idx = inputs.T.reshape(NUM_WORKERS, NCHUNK, CHUNK).astype(jnp.int32)
    gathered = _sc_gather(idx, embedding)
    x = gathered.reshape(WINDOW, BATCH, EMBED)
    w1 = W1.reshape(WINDOW, EMBED, HIDDEN)
    return _mlp(x, w1, b1.reshape(1, HIDDEN), W2, b2.reshape(1, OUT))


# speedup vs baseline: 1.0222x; 1.0052x over previous
import functools

import jax
import jax.numpy as jnp
from jax import lax
from jax.experimental import pallas as pl
from jax.experimental.pallas import tpu as pltpu
from jax.experimental.pallas import tpu_sc as plsc

VOCAB = 100000
EMBED = 128
HIDDEN = 1024
OUT = 45
WINDOW = 7
BATCH = 4096
FLAT = BATCH * WINDOW
NUM_WORKERS = 32
BPW = FLAT // NUM_WORKERS
CHUNK = 128
NCHUNK = BPW // CHUNK

BM = 1024


_sc_mesh = plsc.VectorSubcoreMesh(core_axis_name="c", subcore_axis_name="s")


@functools.partial(
    pl.kernel,
    mesh=_sc_mesh,
    out_type=jax.ShapeDtypeStruct((FLAT, EMBED), jnp.float32),
    scratch_types=[
        pltpu.VMEM((NCHUNK, CHUNK), jnp.int32),
        pltpu.VMEM((BPW, EMBED), jnp.float32),
        pltpu.SemaphoreType.DMA,
        pltpu.SemaphoreType.DMA,
    ],
)
def _sc_gather(idx_hbm, table_hbm, out_hbm, idx_v, rows_v, gsem, osem):
    wid = lax.axis_index("s") * 2 + lax.axis_index("c")
    base = wid * BPW
    pltpu.sync_copy(idx_hbm.at[wid], idx_v)
    gathers = []
    for j in range(NCHUNK):
        gathers.append(
            pltpu.async_copy(
                table_hbm.at[idx_v.at[j]],
                rows_v.at[pl.ds(j * CHUNK, CHUNK)],
                gsem,
            )
        )
    half = 4 * CHUNK
    for cp in gathers[:4]:
        cp.wait()
    o1 = pltpu.async_copy(
        rows_v.at[pl.ds(0, half)], out_hbm.at[pl.ds(base, half)], osem
    )
    for cp in gathers[4:]:
        cp.wait()
    o2 = pltpu.async_copy(
        rows_v.at[pl.ds(half, BPW - half)],
        out_hbm.at[pl.ds(base + half, BPW - half)],
        osem,
    )
    o1.wait()
    o2.wait()


def _mlp_body(x_ref, w1_ref, b1_ref, w2_ref, b2_ref, o_ref, w1bf):
    @pl.when(pl.program_id(0) == 0)
    def _():
        w1bf[...] = w1_ref[...].astype(jnp.bfloat16)

    h = jnp.dot(
        x_ref[0].astype(jnp.bfloat16), w1bf[0],
        preferred_element_type=jnp.float32,
    )
    for p in range(1, WINDOW):
        h = h + jnp.dot(
            x_ref[p].astype(jnp.bfloat16), w1bf[p],
            preferred_element_type=jnp.float32,
        )
    h = jnp.maximum(h + b1_ref[...], 0.0).astype(jnp.bfloat16)
    w2 = w2_ref[...].astype(jnp.bfloat16)
    logits = jnp.dot(h, w2, preferred_element_type=jnp.float32)
    logits = logits + b2_ref[...]
    m = jnp.max(logits, axis=1, keepdims=True)
    lse = jnp.log(jnp.sum(jnp.exp(logits - m), axis=1, keepdims=True)) + m
    o_ref[...] = logits - lse


_mlp = pl.pallas_call(
    _mlp_body,
    grid=(BATCH // BM,),
    in_specs=[
        pl.BlockSpec((WINDOW, BM, EMBED), lambda i: (0, i, 0)),
        pl.BlockSpec((WINDOW, EMBED, HIDDEN), lambda i: (0, 0, 0)),
        pl.BlockSpec((1, HIDDEN), lambda i: (0, 0)),
        pl.BlockSpec((HIDDEN, OUT), lambda i: (0, 0)),
        pl.BlockSpec((1, OUT), lambda i: (0, 0)),
    ],
    out_specs=pl.BlockSpec((BM, OUT), lambda i: (i, 0)),
    out_shape=jax.ShapeDtypeStruct((BATCH, OUT), jnp.float32),
    scratch_shapes=[pltpu.VMEM((WINDOW, EMBED, HIDDEN), jnp.bfloat16)],
)


def kernel(inputs, embedding, W1, b1, W2, b2):
    idx = inputs.T.reshape(NUM_WORKERS, NCHUNK, CHUNK).astype(jnp.int32)
    gathered = _sc_gather(idx, embedding)
    x = gathered.reshape(WINDOW, BATCH, EMBED)
    w1 = W1.reshape(WINDOW, EMBED, HIDDEN)
    return _mlp(x, w1, b1.reshape(1, HIDDEN), W2, b2.reshape(1, OUT))
